# baseline (speedup 1.0000x reference)
import functools

import jax
import jax.numpy as jnp
from jax import lax
from jax.experimental import pallas as pl
from jax.experimental.pallas import tpu as pltpu
from jax.experimental.pallas import tpu_sc as plsc

N = 10000
E = 320000
IN_CH = 128
D = 64

NC = 2
NS = 16
NW = NC * NS
CH = 80
NCHT = E // NS // CH
NCHD = NCHT // NC
RPT = N // NS
NBUF = 5

_MESH = plsc.VectorSubcoreMesh(core_axis_name="c", subcore_axis_name="s")
_SC_PARAMS = pltpu.CompilerParams(use_tc_tiling_on_sc=False)


@functools.partial(
    pl.kernel,
    mesh=_MESH,
    compiler_params=_SC_PARAMS,
    out_type=jax.ShapeDtypeStruct((NC, N, 16), jnp.float32),
    scratch_types=[
        pltpu.VMEM((NCHD, CH), jnp.int32),
        pltpu.VMEM((CH, 16), jnp.float32),
        pltpu.VMEM_SHARED((N, 16), jnp.float32),
        pltpu.SemaphoreType.DMA((NBUF,)),
    ],
)
def _deg_kernel(idx_hbm, ones_hbm, zeros_hbm, out_hbm, dst_v, ones_v, deg_sh,
                sems):
    c = lax.axis_index("c")
    s = lax.axis_index("s")
    pltpu.sync_copy(zeros_hbm.at[s], deg_sh.at[pl.ds(s * RPT, RPT)])
    pltpu.sync_copy(ones_hbm, ones_v)
    pltpu.sync_copy(idx_hbm.at[1].at[s].at[pl.ds(c * NCHD, NCHD)], dst_v)
    plsc.subcore_barrier()

    def outer(kk, carry):
        for b in range(NBUF):
            k = kk * NBUF + b

            @pl.when(k >= NBUF)
            def _():
                pltpu.make_async_copy(ones_v, deg_sh.at[dst_v.at[k - NBUF]],
                                      sems.at[b]).wait()

            pltpu.async_copy(ones_v, deg_sh.at[dst_v.at[k]], sems.at[b],
                             add=True)
        return carry

    lax.fori_loop(0, NCHD // NBUF, outer, 0)
    for b in range(NBUF):
        pltpu.make_async_copy(ones_v, deg_sh.at[dst_v.at[NCHD - NBUF + b]],
                              sems.at[b]).wait()
    plsc.subcore_barrier()
    pltpu.sync_copy(deg_sh.at[pl.ds(s * RPT, RPT)],
                    out_hbm.at[c].at[pl.ds(s * RPT, RPT)])


@functools.partial(
    pl.kernel,
    mesh=_MESH,
    compiler_params=_SC_PARAMS,
    out_type=jax.ShapeDtypeStruct((NC, N, D), jnp.float32),
    scratch_types=[
        pltpu.VMEM((NCHD, CH), jnp.int32),
        pltpu.VMEM((NCHD, CH), jnp.int32),
        pltpu.VMEM((NBUF, CH, D), jnp.float32),
        pltpu.VMEM_SHARED((N, D), jnp.float32),
        pltpu.SemaphoreType.DMA((NBUF,)),
        pltpu.SemaphoreType.DMA((NBUF,)),
    ],
)
def _agg_kernel(rows_hbm, idx_hbm, zeros_hbm, out_hbm,
                src_v, dst_v, rows_v, acc_sh, semg, sems):
    c = lax.axis_index("c")
    s = lax.axis_index("s")
    pltpu.sync_copy(zeros_hbm.at[s], acc_sh.at[pl.ds(s * RPT, RPT)])
    pltpu.sync_copy(idx_hbm.at[0].at[s].at[pl.ds(c * NCHD, NCHD)], src_v)
    pltpu.sync_copy(idx_hbm.at[1].at[s].at[pl.ds(c * NCHD, NCHD)], dst_v)
    plsc.subcore_barrier()

    for b in range(NBUF):
        pltpu.async_copy(rows_hbm.at[src_v.at[b]], rows_v.at[b], semg.at[b])

    def outer(kk, carry):
        for b in range(NBUF):
            k = kk * NBUF + b
            pltpu.make_async_copy(rows_hbm.at[src_v.at[k]], rows_v.at[b],
                                  semg.at[b]).wait()
            pltpu.async_copy(rows_v.at[b], acc_sh.at[dst_v.at[k]], sems.at[b],
                             add=True)
            bb = (b - 4) % NBUF
            kg = k + NBUF - 4

            @pl.when(jnp.logical_and(kg >= NBUF, kg < NCHD))
            def _():
                pltpu.make_async_copy(rows_v.at[bb],
                                      acc_sh.at[dst_v.at[kg - NBUF]],
                                      sems.at[bb]).wait()
                pltpu.async_copy(rows_hbm.at[src_v.at[kg]], rows_v.at[bb],
                                 semg.at[bb])
        return carry

    lax.fori_loop(0, NCHD // NBUF, outer, 0)
    for b in range(NBUF):
        pltpu.make_async_copy(rows_v.at[b],
                              acc_sh.at[dst_v.at[NCHD - NBUF + b]],
                              sems.at[b]).wait()
    plsc.subcore_barrier()
    pltpu.sync_copy(acc_sh.at[pl.ds(s * RPT, RPT)],
                    out_hbm.at[c].at[pl.ds(s * RPT, RPT)])


def _k0_body(x_ref, w1_ref, t1_ref):
    t1_ref[...] = jnp.dot(x_ref[...], w1_ref[...],
                          preferred_element_type=jnp.float32)


def _k1_body(t1_ref, degp_ref, t1p_ref, dinv_ref):
    deg = degp_ref[0, :, 0:1] + degp_ref[1, :, 0:1] + 1.0
    dinv = lax.rsqrt(deg)
    t1p_ref[...] = t1_ref[...] * dinv
    dinv_ref[...] = dinv


def _k3_body(accp_ref, t1p_ref, dinv_ref, b1_ref, hp_ref):
    dinv = dinv_ref[...]
    a = (accp_ref[0] + accp_ref[1] + t1p_ref[...]) * dinv
    h = jnp.maximum(a + b1_ref[...], 0.0)
    hp_ref[...] = h * dinv


def _k5_body(accp_ref, hp_ref, dinv_ref, wmu_ref, bmu_ref, wls_ref, bls_ref,
             mu_ref, ls_ref):
    dinv = dinv_ref[...]
    a2 = (accp_ref[0] + accp_ref[1] + hp_ref[...]) * dinv
    mu_ref[...] = jnp.dot(a2, wmu_ref[...],
                          preferred_element_type=jnp.float32) + bmu_ref[...]
    ls_ref[...] = jnp.dot(a2, wls_ref[...],
                          preferred_element_type=jnp.float32) + bls_ref[...]


_k0 = pl.pallas_call(
    _k0_body,
    out_shape=jax.ShapeDtypeStruct((N, D), jnp.float32),
)

_k1 = pl.pallas_call(
    _k1_body,
    out_shape=[jax.ShapeDtypeStruct((N, D), jnp.float32),
               jax.ShapeDtypeStruct((N, 1), jnp.float32)],
)

_k3 = pl.pallas_call(
    _k3_body,
    out_shape=jax.ShapeDtypeStruct((N, D), jnp.float32),
)

_k5 = pl.pallas_call(
    _k5_body,
    out_shape=[jax.ShapeDtypeStruct((N, D), jnp.float32),
               jax.ShapeDtypeStruct((N, D), jnp.float32)],
)


def kernel(x, edge_index, W1, b1, Wmu, bmu, Wls, bls):
    idx4 = edge_index.reshape(2, NS, NCHT, CH)
    zeros16 = jnp.zeros((NS, RPT, 16), jnp.float32)
    zeros64 = jnp.zeros((NS, RPT, D), jnp.float32)
    ones16 = jnp.ones((CH, 16), jnp.float32)

    t1 = _k0(x, W1)
    degp = _deg_kernel(idx4, ones16, zeros16)
    t1p, dinv = _k1(t1, degp)
    accB = _agg_kernel(t1p, idx4, zeros64)
    hp = _k3(accB, t1p, dinv, b1.reshape(1, D))
    accC = _agg_kernel(hp, idx4, zeros64)
    mu, ls = _k5(accC, hp, dinv, Wmu, bmu.reshape(1, D), Wls, bls.reshape(1, D))
    return (mu, ls)


# speedup vs baseline: 1.1952x; 1.1952x over previous
import functools

import jax
import jax.numpy as jnp
from jax import lax
from jax.experimental import pallas as pl
from jax.experimental.pallas import tpu as pltpu
from jax.experimental.pallas import tpu_sc as plsc

N = 10000
E = 320000
IN_CH = 128
D = 64

NC = 2
NS = 16
NW = NC * NS
CH = 80
NCHT = E // NS // CH
NCHD = NCHT // NC
RPT = N // NS
NBUF = 5

_MESH = plsc.VectorSubcoreMesh(core_axis_name="c", subcore_axis_name="s")
_SC_PARAMS = pltpu.CompilerParams(use_tc_tiling_on_sc=False)


@functools.partial(
    pl.kernel,
    mesh=_MESH,
    compiler_params=_SC_PARAMS,
    out_type=jax.ShapeDtypeStruct((NC, N, 16), jnp.float32),
    scratch_types=[
        pltpu.VMEM((NCHD, CH), jnp.int32),
        pltpu.VMEM((CH, 16), jnp.float32),
        pltpu.VMEM_SHARED((N, 16), jnp.float32),
        pltpu.SemaphoreType.DMA((NBUF,)),
    ],
)
def _deg_kernel(idx_hbm, ones_hbm, zeros_hbm, out_hbm, dst_v, ones_v, deg_sh,
                sems):
    c = lax.axis_index("c")
    s = lax.axis_index("s")
    pltpu.sync_copy(zeros_hbm.at[s], deg_sh.at[pl.ds(s * RPT, RPT)])
    pltpu.sync_copy(ones_hbm, ones_v)
    pltpu.sync_copy(idx_hbm.at[1].at[s].at[pl.ds(c * NCHD, NCHD)], dst_v)
    plsc.subcore_barrier()

    def outer(kk, carry):
        for b in range(NBUF):
            k = kk * NBUF + b

            @pl.when(k >= NBUF)
            def _():
                pltpu.make_async_copy(ones_v, deg_sh.at[dst_v.at[k - NBUF]],
                                      sems.at[b]).wait()

            pltpu.async_copy(ones_v, deg_sh.at[dst_v.at[k]], sems.at[b],
                             add=True)
        return carry

    lax.fori_loop(0, NCHD // NBUF, outer, 0)
    for b in range(NBUF):
        pltpu.make_async_copy(ones_v, deg_sh.at[dst_v.at[NCHD - NBUF + b]],
                              sems.at[b]).wait()
    plsc.subcore_barrier()
    pltpu.sync_copy(deg_sh.at[pl.ds(s * RPT, RPT)],
                    out_hbm.at[c].at[pl.ds(s * RPT, RPT)])


@functools.partial(
    pl.kernel,
    mesh=_MESH,
    compiler_params=_SC_PARAMS,
    out_type=jax.ShapeDtypeStruct((NC, N, D), jnp.float32),
    scratch_types=[
        pltpu.VMEM((NCHD, CH), jnp.int32),
        pltpu.VMEM((NCHD, CH), jnp.int32),
        pltpu.VMEM((NBUF, CH, D), jnp.float32),
        pltpu.VMEM_SHARED((N, D), jnp.float32),
        pltpu.SemaphoreType.DMA((NBUF,)),
        pltpu.SemaphoreType.DMA((NBUF,)),
    ],
)
def _agg_kernel(rows_hbm, idx_hbm, zeros_hbm, out_hbm,
                src_v, dst_v, rows_v, acc_sh, semg, sems):
    c = lax.axis_index("c")
    s = lax.axis_index("s")
    pltpu.sync_copy(zeros_hbm.at[s], acc_sh.at[pl.ds(s * RPT, RPT)])
    pltpu.sync_copy(idx_hbm.at[0].at[s].at[pl.ds(c * NCHD, NCHD)], src_v)
    pltpu.sync_copy(idx_hbm.at[1].at[s].at[pl.ds(c * NCHD, NCHD)], dst_v)
    plsc.subcore_barrier()

    for b in range(NBUF):
        pltpu.async_copy(rows_hbm.at[src_v.at[b]], rows_v.at[b], semg.at[b])

    def outer(kk, carry):
        for b in range(NBUF):
            k = kk * NBUF + b
            pltpu.make_async_copy(rows_hbm.at[src_v.at[k]], rows_v.at[b],
                                  semg.at[b]).wait()
            pltpu.async_copy(rows_v.at[b], acc_sh.at[dst_v.at[k]], sems.at[b],
                             add=True)
            bb = (b - 3) % NBUF
            kg = k + NBUF - 3

            @pl.when(jnp.logical_and(kg >= NBUF, kg < NCHD))
            def _():
                pltpu.make_async_copy(rows_v.at[bb],
                                      acc_sh.at[dst_v.at[kg - NBUF]],
                                      sems.at[bb]).wait()
                pltpu.async_copy(rows_hbm.at[src_v.at[kg]], rows_v.at[bb],
                                 semg.at[bb])
        return carry

    lax.fori_loop(0, NCHD // NBUF, outer, 0)
    for b in range(NBUF):
        pltpu.make_async_copy(rows_v.at[b],
                              acc_sh.at[dst_v.at[NCHD - NBUF + b]],
                              sems.at[b]).wait()
    plsc.subcore_barrier()
    pltpu.sync_copy(acc_sh.at[pl.ds(s * RPT, RPT)],
                    out_hbm.at[c].at[pl.ds(s * RPT, RPT)])


def _k0_body(x_ref, w1_ref, t1_ref):
    t1_ref[...] = jnp.dot(x_ref[...], w1_ref[...],
                          preferred_element_type=jnp.float32)


def _k1_body(t1_ref, degp_ref, t1p_ref, dinv_ref):
    deg = degp_ref[0, :, 0:1] + degp_ref[1, :, 0:1] + 1.0
    dinv = lax.rsqrt(deg)
    t1p_ref[...] = t1_ref[...] * dinv
    dinv_ref[...] = dinv


def _k3_body(accp_ref, t1p_ref, dinv_ref, b1_ref, hp_ref):
    dinv = dinv_ref[...]
    a = (accp_ref[0] + accp_ref[1] + t1p_ref[...]) * dinv
    h = jnp.maximum(a + b1_ref[...], 0.0)
    hp_ref[...] = h * dinv


def _k5_body(accp_ref, hp_ref, dinv_ref, wmu_ref, bmu_ref, wls_ref, bls_ref,
             mu_ref, ls_ref):
    dinv = dinv_ref[...]
    a2 = (accp_ref[0] + accp_ref[1] + hp_ref[...]) * dinv
    mu_ref[...] = jnp.dot(a2, wmu_ref[...],
                          preferred_element_type=jnp.float32) + bmu_ref[...]
    ls_ref[...] = jnp.dot(a2, wls_ref[...],
                          preferred_element_type=jnp.float32) + bls_ref[...]


_k0 = pl.pallas_call(
    _k0_body,
    out_shape=jax.ShapeDtypeStruct((N, D), jnp.float32),
)

_k1 = pl.pallas_call(
    _k1_body,
    out_shape=[jax.ShapeDtypeStruct((N, D), jnp.float32),
               jax.ShapeDtypeStruct((N, 1), jnp.float32)],
)

_k3 = pl.pallas_call(
    _k3_body,
    out_shape=jax.ShapeDtypeStruct((N, D), jnp.float32),
)

_k5 = pl.pallas_call(
    _k5_body,
    out_shape=[jax.ShapeDtypeStruct((N, D), jnp.float32),
               jax.ShapeDtypeStruct((N, D), jnp.float32)],
)


def kernel(x, edge_index, W1, b1, Wmu, bmu, Wls, bls):
    idx4 = edge_index.reshape(2, NS, NCHT, CH)
    zeros16 = jnp.zeros((NS, RPT, 16), jnp.float32)
    zeros64 = jnp.zeros((NS, RPT, D), jnp.float32)
    ones16 = jnp.ones((CH, 16), jnp.float32)

    t1 = _k0(x, W1)
    degp = _deg_kernel(idx4, ones16, zeros16)
    t1p, dinv = _k1(t1, degp)
    accB = _agg_kernel(t1p, idx4, zeros64)
    hp = _k3(accB, t1p, dinv, b1.reshape(1, D))
    accC = _agg_kernel(hp, idx4, zeros64)
    mu, ls = _k5(accC, hp, dinv, Wmu, bmu.reshape(1, D), Wls, bls.reshape(1, D))
    return (mu, ls)


# speedup vs baseline: 1.3419x; 1.1228x over previous
import functools

import jax
import jax.numpy as jnp
from jax import lax
from jax.experimental import pallas as pl
from jax.experimental.pallas import tpu as pltpu
from jax.experimental.pallas import tpu_sc as plsc

N = 10000
E = 320000
IN_CH = 128
D = 64

NC = 2
NS = 16
NW = NC * NS
CH = 80
NCHT = E // NS // CH
NCHD = NCHT // NC
RPT = N // NS
NBUF = 5

_MESH = plsc.VectorSubcoreMesh(core_axis_name="c", subcore_axis_name="s")
_SC_PARAMS = pltpu.CompilerParams(use_tc_tiling_on_sc=False)


@functools.partial(
    pl.kernel,
    mesh=_MESH,
    compiler_params=_SC_PARAMS,
    out_type=jax.ShapeDtypeStruct((NC, N, 16), jnp.float32),
    scratch_types=[
        pltpu.VMEM((NCHD, CH), jnp.int32),
        pltpu.VMEM((CH, 16), jnp.float32),
        pltpu.VMEM_SHARED((N, 16), jnp.float32),
        pltpu.SemaphoreType.DMA((NBUF,)),
    ],
)
def _deg_kernel(idx_hbm, ones_hbm, zeros_hbm, out_hbm, dst_v, ones_v, deg_sh,
                sems):
    c = lax.axis_index("c")
    s = lax.axis_index("s")
    pltpu.sync_copy(zeros_hbm.at[s], deg_sh.at[pl.ds(s * RPT, RPT)])
    pltpu.sync_copy(ones_hbm, ones_v)
    pltpu.sync_copy(idx_hbm.at[1].at[s].at[pl.ds(c * NCHD, NCHD)], dst_v)
    plsc.subcore_barrier()

    def outer(kk, carry):
        for b in range(NBUF):
            k = kk * NBUF + b

            @pl.when(k >= NBUF)
            def _():
                pltpu.make_async_copy(ones_v, deg_sh.at[dst_v.at[k - NBUF]],
                                      sems.at[b]).wait()

            pltpu.async_copy(ones_v, deg_sh.at[dst_v.at[k]], sems.at[b],
                             add=True)
        return carry

    lax.fori_loop(0, NCHD // NBUF, outer, 0)
    for b in range(NBUF):
        pltpu.make_async_copy(ones_v, deg_sh.at[dst_v.at[NCHD - NBUF + b]],
                              sems.at[b]).wait()
    plsc.subcore_barrier()
    pltpu.sync_copy(deg_sh.at[pl.ds(s * RPT, RPT)],
                    out_hbm.at[c].at[pl.ds(s * RPT, RPT)])


@functools.partial(
    pl.kernel,
    mesh=_MESH,
    compiler_params=_SC_PARAMS,
    out_type=jax.ShapeDtypeStruct((NC, N, D), jnp.float32),
    scratch_types=[
        pltpu.VMEM((NCHD, CH), jnp.int32),
        pltpu.VMEM((NCHD, CH), jnp.int32),
        pltpu.VMEM((NBUF, CH, D), jnp.float32),
        pltpu.VMEM_SHARED((N, D), jnp.float32),
        pltpu.SemaphoreType.DMA((NBUF,)),
        pltpu.SemaphoreType.DMA((NBUF,)),
    ],
)
def _agg_kernel(rows_hbm, idx_hbm, zeros_hbm, out_hbm,
                src_v, dst_v, rows_v, acc_sh, semg, sems):
    c = lax.axis_index("c")
    s = lax.axis_index("s")
    pltpu.sync_copy(zeros_hbm.at[s], acc_sh.at[pl.ds(s * RPT, RPT)])
    pltpu.sync_copy(idx_hbm.at[0].at[s].at[pl.ds(c * NCHD, NCHD)], src_v)
    pltpu.sync_copy(idx_hbm.at[1].at[s].at[pl.ds(c * NCHD, NCHD)], dst_v)
    plsc.subcore_barrier()

    for b in range(NBUF):
        pltpu.async_copy(rows_hbm.at[src_v.at[b]], rows_v.at[b], semg.at[b])

    def outer(kk, carry):
        for b in range(NBUF):
            k = kk * NBUF + b
            pltpu.make_async_copy(rows_hbm.at[src_v.at[k]], rows_v.at[b],
                                  semg.at[b]).wait()
            pltpu.async_copy(rows_v.at[b], acc_sh.at[dst_v.at[k]], sems.at[b],
                             add=True)
            bb = (b - 2) % NBUF
            kg = k + NBUF - 2

            @pl.when(jnp.logical_and(kg >= NBUF, kg < NCHD))
            def _():
                pltpu.make_async_copy(rows_v.at[bb],
                                      acc_sh.at[dst_v.at[kg - NBUF]],
                                      sems.at[bb]).wait()
                pltpu.async_copy(rows_hbm.at[src_v.at[kg]], rows_v.at[bb],
                                 semg.at[bb])
        return carry

    lax.fori_loop(0, NCHD // NBUF, outer, 0)
    for b in range(NBUF):
        pltpu.make_async_copy(rows_v.at[b],
                              acc_sh.at[dst_v.at[NCHD - NBUF + b]],
                              sems.at[b]).wait()
    plsc.subcore_barrier()
    pltpu.sync_copy(acc_sh.at[pl.ds(s * RPT, RPT)],
                    out_hbm.at[c].at[pl.ds(s * RPT, RPT)])


def _k0_body(x_ref, w1_ref, t1_ref):
    t1_ref[...] = jnp.dot(x_ref[...], w1_ref[...],
                          preferred_element_type=jnp.float32)


def _k1_body(t1_ref, degp_ref, t1p_ref, dinv_ref):
    deg = degp_ref[0, :, 0:1] + degp_ref[1, :, 0:1] + 1.0
    dinv = lax.rsqrt(deg)
    t1p_ref[...] = t1_ref[...] * dinv
    dinv_ref[...] = dinv


def _k3_body(accp_ref, t1p_ref, dinv_ref, b1_ref, hp_ref):
    dinv = dinv_ref[...]
    a = (accp_ref[0] + accp_ref[1] + t1p_ref[...]) * dinv
    h = jnp.maximum(a + b1_ref[...], 0.0)
    hp_ref[...] = h * dinv


def _k5_body(accp_ref, hp_ref, dinv_ref, wmu_ref, bmu_ref, wls_ref, bls_ref,
             mu_ref, ls_ref):
    dinv = dinv_ref[...]
    a2 = (accp_ref[0] + accp_ref[1] + hp_ref[...]) * dinv
    mu_ref[...] = jnp.dot(a2, wmu_ref[...],
                          preferred_element_type=jnp.float32) + bmu_ref[...]
    ls_ref[...] = jnp.dot(a2, wls_ref[...],
                          preferred_element_type=jnp.float32) + bls_ref[...]


_k0 = pl.pallas_call(
    _k0_body,
    out_shape=jax.ShapeDtypeStruct((N, D), jnp.float32),
)

_k1 = pl.pallas_call(
    _k1_body,
    out_shape=[jax.ShapeDtypeStruct((N, D), jnp.float32),
               jax.ShapeDtypeStruct((N, 1), jnp.float32)],
)

_k3 = pl.pallas_call(
    _k3_body,
    out_shape=jax.ShapeDtypeStruct((N, D), jnp.float32),
)

_k5 = pl.pallas_call(
    _k5_body,
    out_shape=[jax.ShapeDtypeStruct((N, D), jnp.float32),
               jax.ShapeDtypeStruct((N, D), jnp.float32)],
)


def kernel(x, edge_index, W1, b1, Wmu, bmu, Wls, bls):
    idx4 = edge_index.reshape(2, NS, NCHT, CH)
    zeros16 = jnp.zeros((NS, RPT, 16), jnp.float32)
    zeros64 = jnp.zeros((NS, RPT, D), jnp.float32)
    ones16 = jnp.ones((CH, 16), jnp.float32)

    t1 = _k0(x, W1)
    degp = _deg_kernel(idx4, ones16, zeros16)
    t1p, dinv = _k1(t1, degp)
    accB = _agg_kernel(t1p, idx4, zeros64)
    hp = _k3(accB, t1p, dinv, b1.reshape(1, D))
    accC = _agg_kernel(hp, idx4, zeros64)
    mu, ls = _k5(accC, hp, dinv, Wmu, bmu.reshape(1, D), Wls, bls.reshape(1, D))
    return (mu, ls)


# speedup vs baseline: 1.3901x; 1.0359x over previous
import functools

import jax
import jax.numpy as jnp
from jax import lax
from jax.experimental import pallas as pl
from jax.experimental.pallas import tpu as pltpu
from jax.experimental.pallas import tpu_sc as plsc

N = 10000
E = 320000
IN_CH = 128
D = 64

NC = 2
NS = 16
NW = NC * NS
CH = 80
NCHT = E // NS // CH
NCHD = NCHT // NC
RPT = N // NS
NBUF = 5

_MESH = plsc.VectorSubcoreMesh(core_axis_name="c", subcore_axis_name="s")
_SC_PARAMS = pltpu.CompilerParams(use_tc_tiling_on_sc=False)


@functools.partial(
    pl.kernel,
    mesh=_MESH,
    compiler_params=_SC_PARAMS,
    out_type=jax.ShapeDtypeStruct((NC, N, 16), jnp.float32),
    scratch_types=[
        pltpu.VMEM((NCHD, CH), jnp.int32),
        pltpu.VMEM((CH, 16), jnp.float32),
        pltpu.VMEM_SHARED((N, 16), jnp.float32),
        pltpu.SemaphoreType.DMA((NBUF,)),
    ],
)
def _deg_kernel(idx_hbm, out_hbm, dst_v, ones_v, deg_sh, sems):
    c = lax.axis_index("c")
    s = lax.axis_index("s")
    def zrow(r, carry):
        ones_v[r] = jnp.zeros((16,), jnp.float32)
        return carry

    lax.fori_loop(0, CH, zrow, 0)
    for j in range(7):
        pltpu.sync_copy(ones_v, deg_sh.at[pl.ds(s * RPT + j * CH, CH)])
    pltpu.sync_copy(ones_v.at[pl.ds(0, RPT - 7 * CH)],
                    deg_sh.at[pl.ds(s * RPT + 7 * CH, RPT - 7 * CH)])

    def orow(r, carry):
        ones_v[r] = jnp.ones((16,), jnp.float32)
        return carry

    lax.fori_loop(0, CH, orow, 0)
    pltpu.sync_copy(idx_hbm.at[1].at[s].at[pl.ds(c * NCHD, NCHD)], dst_v)
    plsc.subcore_barrier()

    def outer(kk, carry):
        for b in range(NBUF):
            k = kk * NBUF + b

            @pl.when(k >= NBUF)
            def _():
                pltpu.make_async_copy(ones_v, deg_sh.at[dst_v.at[k - NBUF]],
                                      sems.at[b]).wait()

            pltpu.async_copy(ones_v, deg_sh.at[dst_v.at[k]], sems.at[b],
                             add=True)
        return carry

    lax.fori_loop(0, NCHD // NBUF, outer, 0)
    for b in range(NBUF):
        pltpu.make_async_copy(ones_v, deg_sh.at[dst_v.at[NCHD - NBUF + b]],
                              sems.at[b]).wait()
    plsc.subcore_barrier()
    pltpu.sync_copy(deg_sh.at[pl.ds(s * RPT, RPT)],
                    out_hbm.at[c].at[pl.ds(s * RPT, RPT)])


@functools.partial(
    pl.kernel,
    mesh=_MESH,
    compiler_params=_SC_PARAMS,
    out_type=jax.ShapeDtypeStruct((NC, N, D), jnp.float32),
    scratch_types=[
        pltpu.VMEM((NCHD, CH), jnp.int32),
        pltpu.VMEM((NCHD, CH), jnp.int32),
        pltpu.VMEM((NBUF, CH, D), jnp.float32),
        pltpu.VMEM_SHARED((N, D), jnp.float32),
        pltpu.SemaphoreType.DMA((NBUF,)),
        pltpu.SemaphoreType.DMA((NBUF,)),
    ],
)
def _agg_kernel(rows_hbm, idx_hbm, out_hbm,
                src_v, dst_v, rows_v, acc_sh, semg, sems):
    c = lax.axis_index("c")
    s = lax.axis_index("s")

    def zrow(r, carry):
        for j in range(D // 16):
            rows_v[0, r, pl.ds(j * 16, 16)] = jnp.zeros((16,), jnp.float32)
        return carry

    lax.fori_loop(0, CH, zrow, 0)
    for j in range(7):
        pltpu.sync_copy(rows_v.at[0], acc_sh.at[pl.ds(s * RPT + j * CH, CH)])
    pltpu.sync_copy(rows_v.at[0].at[pl.ds(0, RPT - 7 * CH)],
                    acc_sh.at[pl.ds(s * RPT + 7 * CH, RPT - 7 * CH)])
    pltpu.sync_copy(idx_hbm.at[0].at[s].at[pl.ds(c * NCHD, NCHD)], src_v)
    pltpu.sync_copy(idx_hbm.at[1].at[s].at[pl.ds(c * NCHD, NCHD)], dst_v)
    plsc.subcore_barrier()

    for b in range(NBUF):
        pltpu.async_copy(rows_hbm.at[src_v.at[b]], rows_v.at[b], semg.at[b])

    def outer(kk, carry):
        for b in range(NBUF):
            k = kk * NBUF + b
            pltpu.make_async_copy(rows_hbm.at[src_v.at[k]], rows_v.at[b],
                                  semg.at[b]).wait()
            pltpu.async_copy(rows_v.at[b], acc_sh.at[dst_v.at[k]], sems.at[b],
                             add=True)
            bb = (b - 2) % NBUF
            kg = k + NBUF - 2

            @pl.when(jnp.logical_and(kg >= NBUF, kg < NCHD))
            def _():
                pltpu.make_async_copy(rows_v.at[bb],
                                      acc_sh.at[dst_v.at[kg - NBUF]],
                                      sems.at[bb]).wait()
                pltpu.async_copy(rows_hbm.at[src_v.at[kg]], rows_v.at[bb],
                                 semg.at[bb])
        return carry

    lax.fori_loop(0, NCHD // NBUF, outer, 0)
    for b in range(NBUF):
        pltpu.make_async_copy(rows_v.at[b],
                              acc_sh.at[dst_v.at[NCHD - NBUF + b]],
                              sems.at[b]).wait()
    plsc.subcore_barrier()
    pltpu.sync_copy(acc_sh.at[pl.ds(s * RPT, RPT)],
                    out_hbm.at[c].at[pl.ds(s * RPT, RPT)])


def _k0_body(x_ref, w1_ref, t1_ref):
    t1_ref[...] = jnp.dot(x_ref[...], w1_ref[...],
                          preferred_element_type=jnp.float32)


def _k1_body(t1_ref, degp_ref, t1p_ref, dinv_ref):
    deg = degp_ref[0, :, 0:1] + degp_ref[1, :, 0:1] + 1.0
    dinv = lax.rsqrt(deg)
    t1p_ref[...] = t1_ref[...] * dinv
    dinv_ref[...] = dinv


def _k3_body(accp_ref, t1p_ref, dinv_ref, b1_ref, hp_ref):
    dinv = dinv_ref[...]
    a = (accp_ref[0] + accp_ref[1] + t1p_ref[...]) * dinv
    h = jnp.maximum(a + b1_ref[...], 0.0)
    hp_ref[...] = h * dinv


def _k5_body(accp_ref, hp_ref, dinv_ref, wmu_ref, bmu_ref, wls_ref, bls_ref,
             mu_ref, ls_ref):
    dinv = dinv_ref[...]
    a2 = (accp_ref[0] + accp_ref[1] + hp_ref[...]) * dinv
    mu_ref[...] = jnp.dot(a2, wmu_ref[...],
                          preferred_element_type=jnp.float32) + bmu_ref[...]
    ls_ref[...] = jnp.dot(a2, wls_ref[...],
                          preferred_element_type=jnp.float32) + bls_ref[...]


_k0 = pl.pallas_call(
    _k0_body,
    out_shape=jax.ShapeDtypeStruct((N, D), jnp.float32),
)

_k1 = pl.pallas_call(
    _k1_body,
    out_shape=[jax.ShapeDtypeStruct((N, D), jnp.float32),
               jax.ShapeDtypeStruct((N, 1), jnp.float32)],
)

_k3 = pl.pallas_call(
    _k3_body,
    out_shape=jax.ShapeDtypeStruct((N, D), jnp.float32),
)

_k5 = pl.pallas_call(
    _k5_body,
    out_shape=[jax.ShapeDtypeStruct((N, D), jnp.float32),
               jax.ShapeDtypeStruct((N, D), jnp.float32)],
)


def kernel(x, edge_index, W1, b1, Wmu, bmu, Wls, bls):
    idx4 = edge_index.reshape(2, NS, NCHT, CH)

    t1 = _k0(x, W1)
    degp = _deg_kernel(idx4)
    t1p, dinv = _k1(t1, degp)
    accB = _agg_kernel(t1p, idx4)
    hp = _k3(accB, t1p, dinv, b1.reshape(1, D))
    accC = _agg_kernel(hp, idx4)
    mu, ls = _k5(accC, hp, dinv, Wmu, bmu.reshape(1, D), Wls, bls.reshape(1, D))
    return (mu, ls)
